# baseline (speedup 1.0000x reference)
import functools

import jax
import jax.numpy as jnp
from jax import lax
from jax.experimental import pallas as pl
from jax.experimental.pallas import tpu as pltpu
from jax.experimental.pallas import tpu_sc as plsc

_ROWS = 32768
_HID = 1024
_NC = 2
_NS = 16
_BROWS = 32
_SROWS = 512
_NDMA = _ROWS // _NC // _SROWS


def _zero_fill_body(out_hbm, buf, shared, sem):
    sid = lax.axis_index("s")
    cid = lax.axis_index("c")

    def zero_row(r, carry):
        def zero_chunk(c, inner):
            buf[r, pl.ds(c * 16, 16)] = jnp.zeros((16,), jnp.float32)
            return inner

        return lax.fori_loop(0, _HID // 16, zero_chunk, carry)

    lax.fori_loop(0, _BROWS, zero_row, 0)
    pltpu.sync_copy(buf, shared.at[pl.ds(sid * _BROWS, _BROWS), :])
    plsc.subcore_barrier()

    @pl.when(sid == 0)
    def _():
        def fire(j, carry):
            b = cid * 2 + j // 16
            r0 = (j % 16) * _SROWS
            pltpu.make_async_copy(
                shared, out_hbm.at[b, pl.ds(r0, _SROWS), :], sem
            ).start()
            return carry

        lax.fori_loop(0, _NDMA, fire, 0)

        def drain(j, carry):
            pltpu.make_async_copy(
                shared, out_hbm.at[cid * 2, pl.ds(0, _SROWS), :], sem
            ).wait()
            return carry

        lax.fori_loop(0, _NDMA, drain, 0)


_zero_fill = functools.partial(
    pl.kernel,
    out_type=jax.ShapeDtypeStruct((4, _ROWS // 4, _HID), jnp.float32),
    mesh=plsc.VectorSubcoreMesh(core_axis_name="c", subcore_axis_name="s"),
    scratch_types=[
        pltpu.VMEM((_BROWS, _HID), jnp.float32),
        pltpu.VMEM_SHARED((_SROWS, _HID), jnp.float32),
        pltpu.SemaphoreType.DMA,
    ],
)(_zero_fill_body)


def kernel(hidden_states, gate_weight):
    batch, seq, hidden = hidden_states.shape
    rows = batch * seq
    num_experts = gate_weight.shape[0]
    zeros = _zero_fill()
    return zeros, jnp.zeros((rows, num_experts), jnp.float32)


# speedup vs baseline: 1.1564x; 1.1564x over previous
import jax
import jax.numpy as jnp
from jax.experimental import pallas as pl


_BLOCK = 2048


def _moe_gate_kernel(hs_ref, gw_ref, zero_ref, logits_ref):
    zero_ref[...] = jnp.zeros_like(zero_ref)
    logits_ref[...] = jax.lax.dot_general(
        gw_ref[...], hs_ref[...],
        dimension_numbers=(((1,), (1,)), ((), ())),
        preferred_element_type=jnp.float32,
    )


def kernel(hidden_states, gate_weight):
    batch, seq, hidden = hidden_states.shape
    rows = batch * seq
    hs = hidden_states.reshape(rows, hidden)
    num_experts = gate_weight.shape[0]

    zeros, logits_t = pl.pallas_call(
        _moe_gate_kernel,
        grid=(rows // _BLOCK,),
        in_specs=[
            pl.BlockSpec((_BLOCK, hidden), lambda i: (i, 0)),
            pl.BlockSpec((num_experts, hidden), lambda i: (0, 0)),
        ],
        out_specs=[
            pl.BlockSpec((_BLOCK, hidden), lambda i: (i, 0)),
            pl.BlockSpec((num_experts, _BLOCK), lambda i: (0, i)),
        ],
        out_shape=[
            jax.ShapeDtypeStruct((rows, hidden), hidden_states.dtype),
            jax.ShapeDtypeStruct((num_experts, rows), jnp.float32),
        ],
    )(hs, gate_weight)

    return zeros.reshape(batch, seq, hidden), logits_t.T


# speedup vs baseline: 1.1639x; 1.0065x over previous
import jax
import jax.numpy as jnp
from jax.experimental import pallas as pl
from jax.experimental.pallas import tpu as pltpu


_BLOCK = 2048


def _moe_gate_kernel(hs_ref, gw_ref, zero_hbm, logits_ref, zbuf, sem):
    i = pl.program_id(0)
    n = pl.num_programs(0)

    @pl.when(i == 0)
    def _():
        zbuf[...] = jnp.zeros_like(zbuf)

    block = zbuf.shape[0]
    pltpu.make_async_copy(
        zbuf, zero_hbm.at[pl.ds(i * block, block), :], sem
    ).start()

    logits_ref[...] = jax.lax.dot_general(
        gw_ref[...], hs_ref[...],
        dimension_numbers=(((1,), (1,)), ((), ())),
        preferred_element_type=jnp.float32,
    )

    @pl.when(i > 0)
    def _():
        pltpu.make_async_copy(
            zbuf, zero_hbm.at[pl.ds((i - 1) * block, block), :], sem
        ).wait()

    @pl.when(i == n - 1)
    def _():
        pltpu.make_async_copy(
            zbuf, zero_hbm.at[pl.ds(i * block, block), :], sem
        ).wait()


def kernel(hidden_states, gate_weight):
    batch, seq, hidden = hidden_states.shape
    rows = batch * seq
    hs = hidden_states.reshape(rows, hidden)
    num_experts = gate_weight.shape[0]

    zeros, logits_t = pl.pallas_call(
        _moe_gate_kernel,
        grid=(rows // _BLOCK,),
        in_specs=[
            pl.BlockSpec((_BLOCK, hidden), lambda i: (i, 0)),
            pl.BlockSpec((num_experts, hidden), lambda i: (0, 0)),
        ],
        out_specs=[
            pl.BlockSpec(memory_space=pl.ANY),
            pl.BlockSpec((num_experts, _BLOCK), lambda i: (0, i)),
        ],
        out_shape=[
            jax.ShapeDtypeStruct((rows, hidden), hidden_states.dtype),
            jax.ShapeDtypeStruct((num_experts, rows), jnp.float32),
        ],
        scratch_shapes=[
            pltpu.VMEM((_BLOCK, hidden), jnp.float32),
            pltpu.SemaphoreType.DMA,
        ],
    )(hs, gate_weight)

    return zeros.reshape(batch, seq, hidden), logits_t.T
